# baseline (speedup 1.0000x reference)
import itertools

import jax
import jax.numpy as jnp
from jax.experimental import pallas as pl
from jax.experimental.pallas import tpu as pltpu

_AREAS = (4096, 1024, 256, 64, 16)
_OFFS = (0, 4096, 5120, 5376, 5440)
_A = 5456
_NC = 80
_B = 8
_MAXOBJ = 10
_CH = 16
_KC = _NC // _CH


def _fused_kernel(cnt_ref, cp_ref, ct_ref, lp_ref, lt_ref, ind_ref,
                  mask_ref, out_ref, acc_ref):
    k = pl.program_id(0)

    @pl.when(k < _KC)
    def _focal_step():
        red = None
        for c in range(_CH):
            x = cp_ref[:, c, :]
            t = ct_ref[c]
            p = jnp.clip(jax.nn.sigmoid(x), 1e-7, 1.0 - 1e-7)
            lp = jnp.log(p)
            lq = jnp.log(1.0 - p)
            nce = lq + t * (lp - lq)
            na_t = 0.5 * t - 0.75
            tp = p + p - 1.0
            om = p - t * tp
            f = (na_t * nce) * (om * om)
            red = f if red is None else red + f
        acc_ref[...] = jnp.where(k == 0, red, acc_ref[...] + red)

    @pl.when(k == _KC)
    def _dwlm_step():
        pl_, pt_, pr_, pb_ = (lp_ref[:, 0, :], lp_ref[:, 1, :],
                              lp_ref[:, 2, :], lp_ref[:, 3, :])
        tl_, tt_, tr_, tb_ = (lt_ref[:, 0, :], lt_ref[:, 1, :],
                              lt_ref[:, 2, :], lt_ref[:, 3, :])
        area_p = (pl_ + pr_) * (pt_ + pb_)
        area_t = (tl_ + tr_) * (tt_ + tb_)
        iw = jnp.minimum(pl_, tl_) + jnp.minimum(pr_, tr_)
        ih = jnp.minimum(pt_, tt_) + jnp.minimum(pb_, tb_)
        inter = jnp.maximum(iw, 0.0) * jnp.maximum(ih, 0.0)
        union = area_p + area_t - inter + 1e-7
        iou = inter / union
        cw = jnp.maximum(pl_, tl_) + jnp.maximum(pr_, tr_)
        ch = jnp.maximum(pt_, tt_) + jnp.maximum(pb_, tb_)
        area_c = cw * ch + 1e-7
        loc_loss = 1.0 - (iou - (area_c - union) / area_c)

        total = acc_ref[...] + loc_loss
        ind = ind_ref[...]
        cnt = cnt_ref[...]

        out = jnp.zeros_like(total)
        for o in range(_MAXOBJ):
            oh = (ind == o).astype(jnp.float32)
            m = total * oh
            s_cells, c_cells = [], []
            for off, a in zip(_OFFS, _AREAS):
                s_cells.append(
                    jnp.sum(m[:, off:off + a], axis=1, keepdims=True))
                c_cells.append(
                    jnp.sum(oh[:, off:off + a], axis=1, keepdims=True))
            S = jnp.concatenate(s_cells, axis=1)
            C = jnp.concatenate(c_cells, axis=1)

            mean = S / jnp.maximum(1.0, C)
            lmax = jnp.max(mean, axis=1, keepdims=True) + 1e-5
            mean = jnp.where(mean == 0.0, lmax, mean)
            lmin = jnp.min(mean, axis=1, keepdims=True)
            tgt = 1.0 - (mean - lmin) / jnp.maximum(lmax - lmin, 1e-12)

            cols = [tgt[:, i:i + 1] for i in range(5)]
            min_w = None
            for i, j, kk in itertools.combinations(range(5), 3):
                t3 = jnp.minimum(jnp.minimum(cols[i], cols[j]), cols[kk])
                min_w = t3 if min_w is None else jnp.maximum(min_w, t3)
            tgt = jnp.where(tgt >= min_w, tgt, 0.0)
            tgt = tgt * (cnt > o).astype(jnp.float32)

            tmap = jnp.concatenate(
                [jnp.broadcast_to(tgt[:, l:l + 1], (_B, a))
                 for l, a in enumerate(_AREAS)], axis=1)
            out = out + oh * tmap

        mask = mask_ref[...]
        out_ref[...] = jnp.where(mask > 0.0, out, 1.0)


def kernel(cls_pred, loc_pred, cls_tar, loc_tar, ind_tar, bboxes_cnt):
    B = cls_pred.shape[0]
    cp_t = jnp.transpose(cls_pred, (0, 2, 1))
    ct_t = jnp.transpose(cls_tar, (2, 0, 1))
    lp_t = jnp.transpose(loc_pred, (0, 2, 1))
    lt_t = jnp.transpose(loc_tar, (0, 2, 1))
    ind = ind_tar.reshape(B, _A)
    mask = ct_t[_NC + 1]

    out = pl.pallas_call(
        _fused_kernel,
        grid=(_KC + 1,),
        in_specs=[
            pl.BlockSpec((B, 1), lambda k: (0, 0)),
            pl.BlockSpec((B, _CH, _A), lambda k: (0, jnp.minimum(k, _KC - 1), 0)),
            pl.BlockSpec((_CH, B, _A), lambda k: (jnp.minimum(k, _KC - 1), 0, 0)),
            pl.BlockSpec((B, 4, _A), lambda k: (0, 0, 0)),
            pl.BlockSpec((B, 4, _A), lambda k: (0, 0, 0)),
            pl.BlockSpec((B, _A), lambda k: (0, 0)),
            pl.BlockSpec((B, _A), lambda k: (0, 0)),
        ],
        out_specs=pl.BlockSpec((B, _A), lambda k: (0, 0)),
        out_shape=jax.ShapeDtypeStruct((B, _A), jnp.float32),
        scratch_shapes=[pltpu.VMEM((B, _A), jnp.float32)],
    )(bboxes_cnt, cp_t, ct_t, lp_t, lt_t, ind, mask)
    return (out.reshape(B, _A, 1), mask)
